# Initial kernel scaffold; baseline (speedup 1.0000x reference)
import jax
import jax.numpy as jnp
from jax.experimental import pallas as pl


def kernel(hidden_states, Wg_router, W_gate, W_up, W_down):
    raise NotImplementedError("write your pallas kernel here")


# speedup vs baseline: 1.4199x; 1.4199x over previous
import functools

import jax
import jax.numpy as jnp
from jax.experimental import pallas as pl
from jax.experimental.pallas import tpu as pltpu

T = 2048
D = 2048
F = 768
E = 8
BT = 1024


def _router_body(x_ref, wg_ref, rout_ref):
    logits = jnp.dot(x_ref[...], wg_ref[...], preferred_element_type=jnp.float32)
    m = jnp.max(logits, axis=-1, keepdims=True)
    p = jnp.exp(logits - m)
    p = p / jnp.sum(p, axis=-1, keepdims=True)
    lane = jax.lax.broadcasted_iota(jnp.int32, p.shape, 1)
    p1 = jnp.max(p, axis=-1, keepdims=True)
    i1 = jnp.min(jnp.where(p == p1, lane, E), axis=-1, keepdims=True)
    p2m = jnp.where(lane == i1, -1.0, p)
    p2 = jnp.max(p2m, axis=-1, keepdims=True)
    i2 = jnp.min(jnp.where(p2m == p2, lane, E), axis=-1, keepdims=True)
    s = p1 + p2
    rout = jnp.where(lane == i1, p1 / s, 0.0) + jnp.where(lane == i2, p2 / s, 0.0)
    rout_ref[...] = rout


def _ffn_body(x_ref, rout_ref, wg_ref, wu_ref, wd_ref, out_ref):
    e = pl.program_id(1)
    x = x_ref[...]
    g = jnp.dot(x, wg_ref[0], preferred_element_type=jnp.float32)
    u = jnp.dot(x, wu_ref[0], preferred_element_type=jnp.float32)
    h = (g * jax.nn.sigmoid(g) * u).astype(jnp.bfloat16)
    y = jnp.dot(h, wd_ref[0], preferred_element_type=jnp.float32)
    rout = rout_ref[...]
    lane = jax.lax.broadcasted_iota(jnp.int32, rout.shape, 1)
    col = jnp.sum(jnp.where(lane == e, rout, 0.0), axis=-1, keepdims=True)
    contrib = y * col

    @pl.when(e == 0)
    def _init():
        out_ref[...] = contrib

    @pl.when(e > 0)
    def _acc():
        out_ref[...] = out_ref[...] + contrib


@jax.jit
def _moe(hidden_states, Wg_router, W_gate, W_up, W_down):
    routing = pl.pallas_call(
        _router_body,
        out_shape=jax.ShapeDtypeStruct((T, E), jnp.float32),
    )(hidden_states, Wg_router)

    x_bf = hidden_states.astype(jnp.bfloat16)
    wg_bf = W_gate.astype(jnp.bfloat16)
    wu_bf = W_up.astype(jnp.bfloat16)
    wd_bf = W_down.astype(jnp.bfloat16)

    out = pl.pallas_call(
        _ffn_body,
        grid=(T // BT, E),
        in_specs=[
            pl.BlockSpec((BT, D), lambda t, e: (t, 0)),
            pl.BlockSpec((BT, E), lambda t, e: (t, 0)),
            pl.BlockSpec((1, D, F), lambda t, e: (e, 0, 0)),
            pl.BlockSpec((1, D, F), lambda t, e: (e, 0, 0)),
            pl.BlockSpec((1, F, D), lambda t, e: (e, 0, 0)),
        ],
        out_specs=pl.BlockSpec((BT, D), lambda t, e: (t, 0)),
        out_shape=jax.ShapeDtypeStruct((T, D), jnp.float32),
        compiler_params=pltpu.CompilerParams(
            dimension_semantics=("parallel", "arbitrary"),
        ),
    )(x_bf, routing, wg_bf, wu_bf, wd_bf)
    return out


def kernel(hidden_states, Wg_router, W_gate, W_up, W_down):
    return _moe(hidden_states, Wg_router, W_gate, W_up, W_down)
